# baseline (speedup 1.0000x reference)
import functools

import jax
import jax.numpy as jnp
from jax import lax
from jax.experimental import pallas as pl
from jax.experimental.pallas import tpu as pltpu
from jax.experimental.pallas import tpu_sc as plsc

VOCAB = 100000
EMBED = 128
BATCH = 4096
HIST = 50

NC = 2
NS = 16
NW = NC * NS

B_PER_W = BATCH // NW
CHUNK = 8
N_CHUNKS = B_PER_W // CHUNK

_mesh = plsc.VectorSubcoreMesh(core_axis_name="c", subcore_axis_name="s")


@functools.partial(
    pl.kernel,
    out_type=jax.ShapeDtypeStruct((BATCH, HIST, EMBED), jnp.float32),
    mesh=_mesh,
    compiler_params=pltpu.CompilerParams(use_tc_tiling_on_sc=True),
    scratch_types=[
        pltpu.VMEM((B_PER_W * HIST,), jnp.int32),
        pltpu.VMEM((2, CHUNK * HIST, EMBED), jnp.float32),
        pltpu.SemaphoreType.DMA,
        pltpu.SemaphoreType.DMA,
        pltpu.SemaphoreType.DMA,
        pltpu.SemaphoreType.DMA,
    ],
)
def _gather_kernel(idx_hbm, table_hbm, out_hbm, idx_v, rows_v,
                   sem_g0, sem_g1, sem_o0, sem_o1):
    wid = lax.axis_index("s") * NC + lax.axis_index("c")
    base = wid * B_PER_W
    sems_g = (sem_g0, sem_g1)
    sems_o = (sem_o0, sem_o1)
    pltpu.sync_copy(idx_hbm.at[pl.ds(base * HIST, B_PER_W * HIST)], idx_v)

    def gather(c):
        b = c % 2
        return pltpu.async_copy(
            table_hbm.at[idx_v.at[pl.ds(c * CHUNK * HIST, CHUNK * HIST)]],
            rows_v.at[b], sems_g[b])

    def put(c):
        b = c % 2
        return [
            pltpu.async_copy(
                rows_v.at[b].at[pl.ds(i * HIST, HIST)],
                out_hbm.at[base + c * CHUNK + i],
                sems_o[b])
            for i in range(CHUNK)
        ]

    cur = gather(0)
    outs = [None, None]
    for c in range(N_CHUNKS):
        nxt = None
        if c + 1 < N_CHUNKS:
            nb = (c + 1) % 2
            if outs[nb] is not None:
                for d in outs[nb]:
                    d.wait()
            nxt = gather(c + 1)
        cur.wait()
        outs[c % 2] = put(c)
        cur = nxt
    for d in outs[(N_CHUNKS - 1) % 2]:
        d.wait()
    if N_CHUNKS >= 2:
        for d in outs[N_CHUNKS % 2]:
            d.wait()


def kernel(indices, table):
    return _gather_kernel(indices.reshape(-1), table)


# speedup vs baseline: 1.8089x; 1.8089x over previous
import functools

import jax
import jax.numpy as jnp
from jax import lax
from jax.experimental import pallas as pl
from jax.experimental.pallas import tpu as pltpu
from jax.experimental.pallas import tpu_sc as plsc

VOCAB = 100000
EMBED = 128
BATCH = 4096
HIST = 50

NC = 2
NS = 16
NW = NC * NS

B_TOTAL = BATCH * HIST
B_PER_W = B_TOTAL // NW
CHUNK = 400
N_CHUNKS = B_PER_W // CHUNK

_mesh = plsc.VectorSubcoreMesh(core_axis_name="c", subcore_axis_name="s")


@functools.partial(
    pl.kernel,
    out_type=jax.ShapeDtypeStruct((B_TOTAL, EMBED), jnp.float32),
    mesh=_mesh,
    compiler_params=pltpu.CompilerParams(use_tc_tiling_on_sc=True),
    scratch_types=[
        pltpu.VMEM((B_PER_W,), jnp.int32),
        pltpu.VMEM((2, CHUNK, EMBED), jnp.float32),
        pltpu.SemaphoreType.DMA,
        pltpu.SemaphoreType.DMA,
        pltpu.SemaphoreType.DMA,
        pltpu.SemaphoreType.DMA,
    ],
)
def _gather_kernel(idx_hbm, table_hbm, out_hbm, idx_v, rows_v,
                   sem_g0, sem_g1, sem_o0, sem_o1):
    wid = lax.axis_index("s") * NC + lax.axis_index("c")
    base = wid * B_PER_W
    sems_g = (sem_g0, sem_g1)
    sems_o = (sem_o0, sem_o1)
    pltpu.sync_copy(idx_hbm.at[pl.ds(base, B_PER_W)], idx_v)

    def gather(c):
        b = c % 2
        return pltpu.async_copy(
            table_hbm.at[idx_v.at[pl.ds(c * CHUNK, CHUNK)]],
            rows_v.at[b], sems_g[b])

    def put(c):
        b = c % 2
        return pltpu.async_copy(
            rows_v.at[b], out_hbm.at[pl.ds(base + c * CHUNK, CHUNK)],
            sems_o[b])

    cur = gather(0)
    outs = [None, None]
    for c in range(N_CHUNKS):
        nxt = None
        if c + 1 < N_CHUNKS:
            nb = (c + 1) % 2
            if outs[nb] is not None:
                outs[nb].wait()
            nxt = gather(c + 1)
        cur.wait()
        outs[c % 2] = put(c)
        cur = nxt
    outs[(N_CHUNKS - 1) % 2].wait()
    if N_CHUNKS >= 2:
        outs[N_CHUNKS % 2].wait()


def kernel(indices, table):
    flat_t = jnp.transpose(indices).reshape(-1)
    out = _gather_kernel(flat_t, table)
    return jnp.transpose(out.reshape(HIST, BATCH, EMBED), (1, 0, 2))
